# baseline (speedup 1.0000x reference)
import dataclasses

import jax
import jax.numpy as jnp
from jax import lax
from jax.experimental import pallas as pl
from jax.experimental.pallas import tpu as pltpu
from jax.experimental.pallas import tpu_sc as plsc

N_NODES = 10000
N_EDGES = 320000
DIM = 128
NC = 2
NS = 16
NW = NC * NS
CE = 80
NCHUNK = 129
EW = NCHUNK * CE
E_PAD = NW * EW
N_PAD = 10240
RPT = N_PAD // NS
NEG = -1.0e30


def _sc_compiler_params():
    cp = pltpu.CompilerParams()
    if "needs_layout_passes" in pltpu.CompilerParams.__dataclass_fields__:
        cp = dataclasses.replace(cp, needs_layout_passes=False)
    return cp


def _mm_body(h_ref, w_ref, o_ref):
    o_ref[...] = jnp.dot(h_ref[...], w_ref[...],
                         preferred_element_type=jnp.float32)


def _matmul(h, W):
    return pl.pallas_call(
        _mm_body,
        grid=(10,),
        in_specs=[
            pl.BlockSpec((1000, 128), lambda i: (i, 0)),
            pl.BlockSpec((128, 128), lambda i: (0, 0)),
        ],
        out_specs=pl.BlockSpec((1000, 128), lambda i: (i, 0)),
        out_shape=jax.ShapeDtypeStruct((N_NODES, DIM), jnp.float32),
    )(h, W)


def _sc1_body(z_hbm, src_hbm, dst_hbm, e_hbm, mp_hbm,
              src_v, dst_v, ra0, ra1, ra2, rb0, rb1, rb2, e_v, m_v, t0, t1,
              m_sh, sga0, sga1, sga2, sgb0, sgb1, sgb2):
    cidx = lax.axis_index("c")
    sidx = lax.axis_index("s")
    w = cidx * NS + sidx
    nbase = sidx * RPT
    lanes = lax.iota(jnp.int32, 16)
    ras = [ra0, ra1, ra2]
    rbs = [rb0, rb1, rb2]
    sgas = [sga0, sga1, sga2]
    sgbs = [sgb0, sgb1, sgb2]

    pltpu.sync_copy(src_hbm.at[w], src_v)
    pltpu.sync_copy(dst_hbm.at[w], dst_v)

    @pl.loop(0, N_PAD, step=16)
    def _(i):
        m_v[pl.ds(i, 16)] = jnp.full((16,), NEG, jnp.float32)

    def issue(c, b):
        pltpu.make_async_copy(z_hbm.at[src_v.at[c]], ras[b], sgas[b]).start()
        pltpu.make_async_copy(z_hbm.at[dst_v.at[c]], rbs[b], sgbs[b]).start()

    issue(0, 0)
    issue(1, 1)

    @pl.loop(0, NCHUNK, step=3)
    def _(c0):
        for u in range(3):
            c = c0 + u
            rows_a, rows_b = ras[u], rbs[u]
            pltpu.make_async_copy(z_hbm.at[src_v.at[c]], rows_a,
                                  sgas[u]).wait()
            pltpu.make_async_copy(z_hbm.at[dst_v.at[c]], rows_b,
                                  sgbs[u]).wait()

            @pl.when(c + 2 < NCHUNK)
            def _():
                issue(c + 2, (u + 2) % 3)

            for g in range(CE // 16):
                @pl.loop(0, 16, init_carry=jnp.zeros((16,), jnp.float32),
                         unroll=4)
                def evec(l, ev_c):
                    j = g * 16 + l
                    acc = rows_a[j, pl.ds(0, 16)] * rows_b[j, pl.ds(0, 16)]
                    for k in range(1, 8):
                        acc = acc + (rows_a[j, pl.ds(k * 16, 16)]
                                     * rows_b[j, pl.ds(k * 16, 16)])
                    sv = jnp.sum(acc)
                    return jnp.where(lanes == l, sv, ev_c)

                evec = jnp.where(evec >= 0, evec, evec * jnp.float32(0.2))
                e_v[pl.ds(c * CE + g * 16, 16)] = evec

                dstg = dst_v[c, pl.ds(g * 16, 16)]
                ev = evec
                for l in range(16):
                    cur = plsc.load_gather(m_v, [dstg])
                    plsc.store_scatter(m_v, [dstg], jnp.maximum(cur, ev),
                                       mask=lanes == l)

    pltpu.sync_copy(m_v, m_sh.at[pl.ds(sidx * N_PAD, N_PAD)])
    plsc.subcore_barrier()
    pltpu.sync_copy(m_sh.at[pl.ds(nbase, RPT)], t0)
    for ss in range(1, NS):
        pltpu.sync_copy(m_sh.at[pl.ds(ss * N_PAD + nbase, RPT)], t1)

        @pl.loop(0, RPT, step=16)
        def _(i):
            t0[pl.ds(i, 16)] = jnp.maximum(t0[pl.ds(i, 16)],
                                           t1[pl.ds(i, 16)])
    pltpu.sync_copy(t0, m_sh.at[pl.ds(NS * N_PAD + nbase, RPT)])
    plsc.subcore_barrier()
    pltpu.sync_copy(m_sh.at[pl.ds(NS * N_PAD, N_PAD)], m_v)

    @pl.when(sidx == 0)
    def _():
        pltpu.sync_copy(m_sh.at[pl.ds(NS * N_PAD, N_PAD)], mp_hbm.at[cidx])

    @pl.loop(0, NCHUNK)
    def _(c):
        for g in range(CE // 16):
            dstg = dst_v[c, pl.ds(g * 16, 16)]
            base = c * CE + g * 16
            mg = plsc.load_gather(m_v, [dstg])
            e_v[pl.ds(base, 16)] = e_v[pl.ds(base, 16)] - mg

    pltpu.sync_copy(e_v, e_hbm.at[w])


def _sc1(z_pad, src, dst):
    mesh = plsc.VectorSubcoreMesh(core_axis_name="c", subcore_axis_name="s")
    f = pl.kernel(
        _sc1_body,
        out_type=(
            jax.ShapeDtypeStruct((NW, EW), jnp.float32),
            jax.ShapeDtypeStruct((NC, N_PAD), jnp.float32),
        ),
        mesh=mesh,
        scratch_types=[
            pltpu.VMEM((NCHUNK, CE), jnp.int32),
            pltpu.VMEM((NCHUNK, CE), jnp.int32),
            pltpu.VMEM((CE, DIM), jnp.float32),
            pltpu.VMEM((CE, DIM), jnp.float32),
            pltpu.VMEM((CE, DIM), jnp.float32),
            pltpu.VMEM((CE, DIM), jnp.float32),
            pltpu.VMEM((CE, DIM), jnp.float32),
            pltpu.VMEM((CE, DIM), jnp.float32),
            pltpu.VMEM((EW,), jnp.float32),
            pltpu.VMEM((N_PAD,), jnp.float32),
            pltpu.VMEM((RPT,), jnp.float32),
            pltpu.VMEM((RPT,), jnp.float32),
            pltpu.VMEM_SHARED(((NS + 1) * N_PAD,), jnp.float32),
            pltpu.SemaphoreType.DMA,
            pltpu.SemaphoreType.DMA,
            pltpu.SemaphoreType.DMA,
            pltpu.SemaphoreType.DMA,
            pltpu.SemaphoreType.DMA,
            pltpu.SemaphoreType.DMA,
        ],
        compiler_params=_sc_compiler_params(),
    )
    return f(z_pad, src, dst)


def _sc2_body(z_hbm, src_hbm, dst_hbm, e_hbm, hp_hbm, sp_hbm,
              sc0, sc1, sc2, dc0, dc1, dc2, ec0, ec1, ec2,
              pb0, pb1, pb2, r0, r1, r2, t0, s_sh, h_sh,
              sg0, sg1, sg2, ss0, ss1, ss2, sq0, sq1, sq2):
    cidx = lax.axis_index("c")
    sidx = lax.axis_index("s")
    w = cidx * NS + sidx
    nbase = sidx * RPT
    scs = [sc0, sc1, sc2]
    dcs = [dc0, dc1, dc2]
    ecs = [ec0, ec1, ec2]
    pbs = [pb0, pb1, pb2]
    rows = [r0, r1, r2]
    sgs = [sg0, sg1, sg2]
    sss = [ss0, ss1, ss2]
    sqs = [sq0, sq1, sq2]

    @pl.loop(0, CE)
    def _(j):
        for k in range(8):
            r0[j, pl.ds(k * 16, 16)] = jnp.zeros((16,), jnp.float32)

    @pl.loop(0, RPT, step=16)
    def _(i):
        t0[pl.ds(i, 16)] = jnp.zeros((16,), jnp.float32)

    for k in range(8):
        pltpu.sync_copy(r0, h_sh.at[pl.ds(nbase + k * CE, CE), :])
    pltpu.sync_copy(t0, s_sh.at[pl.ds(nbase, RPT)])
    plsc.subcore_barrier()

    def prep(c, b):
        pltpu.sync_copy(src_hbm.at[w, c], scs[b])
        pltpu.sync_copy(dst_hbm.at[w, c], dcs[b])
        pltpu.sync_copy(e_hbm.at[w, c], ecs[b])
        pltpu.make_async_copy(z_hbm.at[scs[b].at[0]], rows[b],
                              sgs[b]).start()

    prep(0, 0)
    prep(1, 1)

    @pl.loop(0, NCHUNK, step=3)
    def _(c0):
        for u in range(3):
            c = c0 + u
            pn = (u + 2) % 3
            rw, pb, dc = rows[u], pbs[u], dcs[u]
            pltpu.make_async_copy(z_hbm.at[scs[u].at[0]], rw, sgs[u]).wait()
            for g in range(CE // 16):
                sl = pl.ds(g * 16, 16)
                pb[0, sl] = jnp.exp(ecs[u][0, sl])

            @pl.loop(0, CE, unroll=4)
            def _(j):
                psc = pb[0, pl.ds(j, 16)][0]
                for k in range(8):
                    rw[j, pl.ds(k * 16, 16)] = rw[j, pl.ds(k * 16, 16)] * psc

            pltpu.async_copy(rw, h_sh.at[dc.at[0]], sss[u], add=True)
            pltpu.async_copy(pb.at[0, pl.ds(0, CE)], s_sh.at[dc.at[0]],
                             sqs[u], add=True)

            @pl.when(c + 2 < NCHUNK)
            def _():
                @pl.when(c >= 1)
                def _():
                    pltpu.make_async_copy(rows[pn],
                                          h_sh.at[pl.ds(0, CE), :],
                                          sss[pn]).wait()
                    pltpu.make_async_copy(pbs[pn].at[0, pl.ds(0, CE)],
                                          s_sh.at[pl.ds(0, CE)],
                                          sqs[pn]).wait()
                prep(c + 2, pn)

    for b in range(3):
        pltpu.make_async_copy(rows[b], h_sh.at[pl.ds(0, CE), :],
                              sss[b]).wait()
        pltpu.make_async_copy(pbs[b].at[0, pl.ds(0, CE)],
                              s_sh.at[pl.ds(0, CE)], sqs[b]).wait()
    plsc.subcore_barrier()
    pltpu.sync_copy(h_sh.at[pl.ds(nbase, RPT), :],
                    hp_hbm.at[cidx, pl.ds(nbase, RPT), :])
    pltpu.sync_copy(s_sh.at[pl.ds(nbase, RPT)],
                    sp_hbm.at[cidx, pl.ds(nbase, RPT)])


def _sc2(z_pad, src4, dst4, e4):
    mesh = plsc.VectorSubcoreMesh(core_axis_name="c", subcore_axis_name="s")
    f = pl.kernel(
        _sc2_body,
        out_type=(
            jax.ShapeDtypeStruct((NC, N_PAD, DIM), jnp.float32),
            jax.ShapeDtypeStruct((NC, N_PAD), jnp.float32),
        ),
        mesh=mesh,
        scratch_types=(
            [pltpu.VMEM((1, CE), jnp.int32) for _ in range(3)]
            + [pltpu.VMEM((1, CE), jnp.int32) for _ in range(3)]
            + [pltpu.VMEM((1, CE), jnp.float32) for _ in range(3)]
            + [pltpu.VMEM((1, CE + 16), jnp.float32) for _ in range(3)]
            + [pltpu.VMEM((CE, DIM), jnp.float32) for _ in range(3)]
            + [pltpu.VMEM((RPT,), jnp.float32)]
            + [pltpu.VMEM_SHARED((N_PAD,), jnp.float32)]
            + [pltpu.VMEM_SHARED((N_PAD, DIM), jnp.float32)]
            + [pltpu.SemaphoreType.DMA for _ in range(9)]
        ),
        compiler_params=_sc_compiler_params(),
    )
    return f(z_pad, src4, dst4, e4)


def _fin_body(hp_ref, sp_ref, mp_ref, o_ref):
    m0 = mp_ref[:, 0:1]
    m1 = mp_ref[:, 1:2]
    mg = jnp.maximum(m0, m1)
    w0 = jnp.exp(m0 - mg)
    w1 = jnp.exp(m1 - mg)
    num = hp_ref[0] * w0 + hp_ref[1] * w1
    den = sp_ref[:, 0:1] * w0 + sp_ref[:, 1:2] * w1
    ok = den > 0
    r = jnp.where(ok, num / jnp.where(ok, den, jnp.float32(1.0)),
                  jnp.float32(0.0))
    neg = jnp.exp(jnp.minimum(r, jnp.float32(0.0))) - jnp.float32(1.0)
    o_ref[...] = jnp.where(r > 0, r, neg)


def _finalize(h_part, s_t, m_t):
    blk = 256
    return pl.pallas_call(
        _fin_body,
        grid=(N_PAD // blk,),
        in_specs=[
            pl.BlockSpec((NC, blk, DIM), lambda i: (0, i, 0)),
            pl.BlockSpec((blk, NC), lambda i: (i, 0)),
            pl.BlockSpec((blk, NC), lambda i: (i, 0)),
        ],
        out_specs=pl.BlockSpec((blk, DIM), lambda i: (i, 0)),
        out_shape=jax.ShapeDtypeStruct((N_PAD, DIM), jnp.float32),
    )(h_part, s_t, m_t)


def kernel(h, edge_index, W):
    z = _matmul(h, W)
    z_pad = jnp.concatenate(
        [z, jnp.zeros((N_PAD - N_NODES, DIM), jnp.float32)], axis=0)
    npad = E_PAD - N_EDGES
    fill = (N_NODES
            + jnp.arange(npad, dtype=jnp.int32) % (N_PAD - N_NODES))
    src = jnp.concatenate([edge_index[0].astype(jnp.int32), fill])
    dst = jnp.concatenate([edge_index[1].astype(jnp.int32), fill])
    src = src.reshape(NW, NCHUNK, CE)
    dst = dst.reshape(NW, NCHUNK, CE)
    e, m_part = _sc1(z_pad, src, dst)
    h_part, s_part = _sc2(z_pad, src.reshape(NW, NCHUNK, 1, CE),
                          dst.reshape(NW, NCHUNK, 1, CE),
                          e.reshape(NW, NCHUNK, 1, CE))
    out = _finalize(h_part, s_part.T, m_part.T)
    return out[:N_NODES]


# speedup vs baseline: 1.2086x; 1.2086x over previous
import dataclasses

import jax
import jax.numpy as jnp
from jax import lax
from jax.experimental import pallas as pl
from jax.experimental.pallas import tpu as pltpu
from jax.experimental.pallas import tpu_sc as plsc

N_NODES = 10000
N_EDGES = 320000
DIM = 128
NC = 2
NS = 16
NW = NC * NS
CE = 80
NCHUNK = 129
EW = NCHUNK * CE
E_PAD = NW * EW
N_PAD = 10240
RPT = N_PAD // NS
NEG = -1.0e30


def _sc_compiler_params():
    cp = pltpu.CompilerParams()
    if "needs_layout_passes" in pltpu.CompilerParams.__dataclass_fields__:
        cp = dataclasses.replace(cp, needs_layout_passes=False)
    return cp


def _mm_body(h_ref, w_ref, o_ref):
    o_ref[...] = jnp.dot(h_ref[...], w_ref[...],
                         preferred_element_type=jnp.float32)


def _matmul(h, W):
    return pl.pallas_call(
        _mm_body,
        grid=(10,),
        in_specs=[
            pl.BlockSpec((1000, 128), lambda i: (i, 0)),
            pl.BlockSpec((128, 128), lambda i: (0, 0)),
        ],
        out_specs=pl.BlockSpec((1000, 128), lambda i: (i, 0)),
        out_shape=jax.ShapeDtypeStruct((N_NODES, DIM), jnp.float32),
    )(h, W)


def _sc1_body(z_hbm, src_hbm, dst_hbm, e_hbm, mp_hbm,
              src_v, dst_v, ra0, ra1, ra2, rb0, rb1, rb2, e_v, m_v, t0, t1,
              m_sh, sga0, sga1, sga2, sgb0, sgb1, sgb2):
    cidx = lax.axis_index("c")
    sidx = lax.axis_index("s")
    w = cidx * NS + sidx
    nbase = sidx * RPT
    lanes = lax.iota(jnp.int32, 16)
    ras = [ra0, ra1, ra2]
    rbs = [rb0, rb1, rb2]
    sgas = [sga0, sga1, sga2]
    sgbs = [sgb0, sgb1, sgb2]

    pltpu.sync_copy(src_hbm.at[w], src_v)
    pltpu.sync_copy(dst_hbm.at[w], dst_v)

    @pl.loop(0, N_PAD, step=16)
    def _(i):
        m_v[pl.ds(i, 16)] = jnp.full((16,), NEG, jnp.float32)

    def issue(c, b):
        pltpu.make_async_copy(z_hbm.at[src_v.at[c]], ras[b], sgas[b]).start()
        pltpu.make_async_copy(z_hbm.at[dst_v.at[c]], rbs[b], sgbs[b]).start()

    issue(0, 0)
    issue(1, 1)

    @pl.loop(0, NCHUNK, step=3)
    def _(c0):
        for u in range(3):
            c = c0 + u
            rows_a, rows_b = ras[u], rbs[u]
            pltpu.make_async_copy(z_hbm.at[src_v.at[c]], rows_a,
                                  sgas[u]).wait()
            pltpu.make_async_copy(z_hbm.at[dst_v.at[c]], rows_b,
                                  sgbs[u]).wait()

            @pl.when(c + 2 < NCHUNK)
            def _():
                issue(c + 2, (u + 2) % 3)

            for g in range(CE // 16):
                @pl.loop(0, 16, init_carry=jnp.zeros((16,), jnp.float32),
                         unroll=4)
                def evec(l, ev_c):
                    j = g * 16 + l
                    acc = rows_a[j, pl.ds(0, 16)] * rows_b[j, pl.ds(0, 16)]
                    for k in range(1, 8):
                        acc = acc + (rows_a[j, pl.ds(k * 16, 16)]
                                     * rows_b[j, pl.ds(k * 16, 16)])
                    sv = jnp.sum(acc)
                    return jnp.where(lanes == l, sv, ev_c)

                evec = jnp.where(evec >= 0, evec, evec * jnp.float32(0.2))
                e_v[pl.ds(c * CE + g * 16, 16)] = evec

                dstg = dst_v[c, pl.ds(g * 16, 16)]
                ev = evec
                for l in range(16):
                    cur = plsc.load_gather(m_v, [dstg])
                    plsc.store_scatter(m_v, [dstg], jnp.maximum(cur, ev),
                                       mask=lanes == l)

    pltpu.sync_copy(m_v, m_sh.at[pl.ds(sidx * N_PAD, N_PAD)])
    plsc.subcore_barrier()
    pltpu.sync_copy(m_sh.at[pl.ds(nbase, RPT)], t0)
    for ss in range(1, NS):
        pltpu.sync_copy(m_sh.at[pl.ds(ss * N_PAD + nbase, RPT)], t1)

        @pl.loop(0, RPT, step=16)
        def _(i):
            t0[pl.ds(i, 16)] = jnp.maximum(t0[pl.ds(i, 16)],
                                           t1[pl.ds(i, 16)])
    pltpu.sync_copy(t0, m_sh.at[pl.ds(NS * N_PAD + nbase, RPT)])
    plsc.subcore_barrier()
    pltpu.sync_copy(m_sh.at[pl.ds(NS * N_PAD, N_PAD)], m_v)

    @pl.when(sidx == 0)
    def _():
        pltpu.sync_copy(m_sh.at[pl.ds(NS * N_PAD, N_PAD)], mp_hbm.at[cidx])

    @pl.loop(0, NCHUNK)
    def _(c):
        for g in range(CE // 16):
            dstg = dst_v[c, pl.ds(g * 16, 16)]
            base = c * CE + g * 16
            mg = plsc.load_gather(m_v, [dstg])
            e_v[pl.ds(base, 16)] = e_v[pl.ds(base, 16)] - mg

    pltpu.sync_copy(e_v, e_hbm.at[w])


def _sc1(z_pad, src, dst):
    mesh = plsc.VectorSubcoreMesh(core_axis_name="c", subcore_axis_name="s")
    f = pl.kernel(
        _sc1_body,
        out_type=(
            jax.ShapeDtypeStruct((NW, EW), jnp.float32),
            jax.ShapeDtypeStruct((NC, N_PAD), jnp.float32),
        ),
        mesh=mesh,
        scratch_types=[
            pltpu.VMEM((NCHUNK, CE), jnp.int32),
            pltpu.VMEM((NCHUNK, CE), jnp.int32),
            pltpu.VMEM((CE, DIM), jnp.float32),
            pltpu.VMEM((CE, DIM), jnp.float32),
            pltpu.VMEM((CE, DIM), jnp.float32),
            pltpu.VMEM((CE, DIM), jnp.float32),
            pltpu.VMEM((CE, DIM), jnp.float32),
            pltpu.VMEM((CE, DIM), jnp.float32),
            pltpu.VMEM((EW,), jnp.float32),
            pltpu.VMEM((N_PAD,), jnp.float32),
            pltpu.VMEM((RPT,), jnp.float32),
            pltpu.VMEM((RPT,), jnp.float32),
            pltpu.VMEM_SHARED(((NS + 1) * N_PAD,), jnp.float32),
            pltpu.SemaphoreType.DMA,
            pltpu.SemaphoreType.DMA,
            pltpu.SemaphoreType.DMA,
            pltpu.SemaphoreType.DMA,
            pltpu.SemaphoreType.DMA,
            pltpu.SemaphoreType.DMA,
        ],
        compiler_params=_sc_compiler_params(),
    )
    return f(z_pad, src, dst)


def _sc2_body(z_hbm, src_hbm, dst_hbm, e_hbm, hp_hbm, sp_hbm,
              sc0, sc1, sc2, dc0, dc1, dc2, ec0, ec1, ec2,
              pb0, pb1, pb2, r0, r1, r2, t0, s_sh, h_sh,
              sg0, sg1, sg2, ss0, ss1, ss2, sq0, sq1, sq2,
              si0, si1, si2):
    cidx = lax.axis_index("c")
    sidx = lax.axis_index("s")
    w = cidx * NS + sidx
    nbase = sidx * RPT
    scs = [sc0, sc1, sc2]
    dcs = [dc0, dc1, dc2]
    ecs = [ec0, ec1, ec2]
    pbs = [pb0, pb1, pb2]
    rows = [r0, r1, r2]
    sgs = [sg0, sg1, sg2]
    sss = [ss0, ss1, ss2]
    sqs = [sq0, sq1, sq2]
    sis = [si0, si1, si2]

    @pl.loop(0, CE)
    def _(j):
        for k in range(8):
            r0[j, pl.ds(k * 16, 16)] = jnp.zeros((16,), jnp.float32)

    @pl.loop(0, RPT, step=16)
    def _(i):
        t0[pl.ds(i, 16)] = jnp.zeros((16,), jnp.float32)

    for k in range(8):
        pltpu.sync_copy(r0, h_sh.at[pl.ds(nbase + k * CE, CE), :])
    pltpu.sync_copy(t0, s_sh.at[pl.ds(nbase, RPT)])
    plsc.subcore_barrier()

    def prep_idx(c, b):
        pltpu.make_async_copy(src_hbm.at[w, c], scs[b], sis[b]).start()
        pltpu.make_async_copy(dst_hbm.at[w, c], dcs[b], sis[b]).start()
        pltpu.make_async_copy(e_hbm.at[w, c], ecs[b], sis[b]).start()

    def wait_idx(c, b):
        pltpu.make_async_copy(src_hbm.at[w, c], scs[b], sis[b]).wait()
        pltpu.make_async_copy(dst_hbm.at[w, c], dcs[b], sis[b]).wait()
        pltpu.make_async_copy(e_hbm.at[w, c], ecs[b], sis[b]).wait()

    prep_idx(0, 0)
    prep_idx(1, 1)
    wait_idx(0, 0)
    pltpu.make_async_copy(z_hbm.at[scs[0].at[0]], rows[0], sgs[0]).start()

    @pl.loop(0, NCHUNK, step=3)
    def _(c0):
        for u in range(3):
            c = c0 + u
            un = (u + 1) % 3
            pn = (u + 2) % 3
            rw, pb, dc = rows[u], pbs[u], dcs[u]

            @pl.when(c + 1 < NCHUNK)
            def _():
                wait_idx(c + 1, un)
                pltpu.make_async_copy(z_hbm.at[scs[un].at[0]], rows[un],
                                      sgs[un]).start()

            pltpu.make_async_copy(z_hbm.at[scs[u].at[0]], rw, sgs[u]).wait()
            for g in range(CE // 16):
                sl = pl.ds(g * 16, 16)
                pb[0, sl] = jnp.exp(ecs[u][0, sl])

            @pl.loop(0, CE, unroll=4)
            def _(j):
                psc = pb[0, pl.ds(j, 16)][0]
                for k in range(8):
                    rw[j, pl.ds(k * 16, 16)] = rw[j, pl.ds(k * 16, 16)] * psc

            pltpu.async_copy(rw, h_sh.at[dc.at[0]], sss[u], add=True)
            pltpu.async_copy(pb.at[0, pl.ds(0, CE)], s_sh.at[dc.at[0]],
                             sqs[u], add=True)

            @pl.when(c + 2 < NCHUNK)
            def _():
                @pl.when(c >= 1)
                def _():
                    pltpu.make_async_copy(rows[pn],
                                          h_sh.at[pl.ds(0, CE), :],
                                          sss[pn]).wait()
                    pltpu.make_async_copy(pbs[pn].at[0, pl.ds(0, CE)],
                                          s_sh.at[pl.ds(0, CE)],
                                          sqs[pn]).wait()
                prep_idx(c + 2, pn)

    for b in range(3):
        pltpu.make_async_copy(rows[b], h_sh.at[pl.ds(0, CE), :],
                              sss[b]).wait()
        pltpu.make_async_copy(pbs[b].at[0, pl.ds(0, CE)],
                              s_sh.at[pl.ds(0, CE)], sqs[b]).wait()
    plsc.subcore_barrier()
    pltpu.sync_copy(h_sh.at[pl.ds(nbase, RPT), :],
                    hp_hbm.at[cidx, pl.ds(nbase, RPT), :])
    pltpu.sync_copy(s_sh.at[pl.ds(nbase, RPT)],
                    sp_hbm.at[cidx, pl.ds(nbase, RPT)])


def _sc2(z_pad, src4, dst4, e4):
    mesh = plsc.VectorSubcoreMesh(core_axis_name="c", subcore_axis_name="s")
    f = pl.kernel(
        _sc2_body,
        out_type=(
            jax.ShapeDtypeStruct((NC, N_PAD, DIM), jnp.float32),
            jax.ShapeDtypeStruct((NC, N_PAD), jnp.float32),
        ),
        mesh=mesh,
        scratch_types=(
            [pltpu.VMEM((1, CE), jnp.int32) for _ in range(3)]
            + [pltpu.VMEM((1, CE), jnp.int32) for _ in range(3)]
            + [pltpu.VMEM((1, CE), jnp.float32) for _ in range(3)]
            + [pltpu.VMEM((1, CE + 16), jnp.float32) for _ in range(3)]
            + [pltpu.VMEM((CE, DIM), jnp.float32) for _ in range(3)]
            + [pltpu.VMEM((RPT,), jnp.float32)]
            + [pltpu.VMEM_SHARED((N_PAD,), jnp.float32)]
            + [pltpu.VMEM_SHARED((N_PAD, DIM), jnp.float32)]
            + [pltpu.SemaphoreType.DMA for _ in range(12)]
        ),
        compiler_params=_sc_compiler_params(),
    )
    return f(z_pad, src4, dst4, e4)


def _fin_body(hp_ref, sp_ref, mp_ref, o_ref):
    m0 = mp_ref[:, 0:1]
    m1 = mp_ref[:, 1:2]
    mg = jnp.maximum(m0, m1)
    w0 = jnp.exp(m0 - mg)
    w1 = jnp.exp(m1 - mg)
    num = hp_ref[0] * w0 + hp_ref[1] * w1
    den = sp_ref[:, 0:1] * w0 + sp_ref[:, 1:2] * w1
    ok = den > 0
    r = jnp.where(ok, num / jnp.where(ok, den, jnp.float32(1.0)),
                  jnp.float32(0.0))
    neg = jnp.exp(jnp.minimum(r, jnp.float32(0.0))) - jnp.float32(1.0)
    o_ref[...] = jnp.where(r > 0, r, neg)


def _finalize(h_part, s_t, m_t):
    blk = 256
    return pl.pallas_call(
        _fin_body,
        grid=(N_PAD // blk,),
        in_specs=[
            pl.BlockSpec((NC, blk, DIM), lambda i: (0, i, 0)),
            pl.BlockSpec((blk, NC), lambda i: (i, 0)),
            pl.BlockSpec((blk, NC), lambda i: (i, 0)),
        ],
        out_specs=pl.BlockSpec((blk, DIM), lambda i: (i, 0)),
        out_shape=jax.ShapeDtypeStruct((N_PAD, DIM), jnp.float32),
    )(h_part, s_t, m_t)


def kernel(h, edge_index, W):
    z = _matmul(h, W)
    z_pad = jnp.concatenate(
        [z, jnp.zeros((N_PAD - N_NODES, DIM), jnp.float32)], axis=0)
    npad = E_PAD - N_EDGES
    fill = (N_NODES
            + jnp.arange(npad, dtype=jnp.int32) % (N_PAD - N_NODES))
    src = jnp.concatenate([edge_index[0].astype(jnp.int32), fill])
    dst = jnp.concatenate([edge_index[1].astype(jnp.int32), fill])
    src = src.reshape(NW, NCHUNK, CE)
    dst = dst.reshape(NW, NCHUNK, CE)
    e, m_part = _sc1(z_pad, src, dst)
    h_part, s_part = _sc2(z_pad, src.reshape(NW, NCHUNK, 1, CE),
                          dst.reshape(NW, NCHUNK, 1, CE),
                          e.reshape(NW, NCHUNK, 1, CE))
    out = _finalize(h_part, s_part.T, m_part.T)
    return out[:N_NODES]
